# baseline (speedup 1.0000x reference)
import jax
import jax.numpy as jnp
from jax import lax
from jax.experimental import pallas as pl
from jax.experimental.pallas import tpu as pltpu
from jax.experimental.pallas import tpu_sc as plsc

NC = 2
NS = 16
NW = NC * NS
BATCH = 16384
BPW = BATCH // NW
CHUNK = 128
NCHUNK = BPW // CHUNK
D = 4
L = 16
NROW = NCHUNK * D


def _rsqrt_clamped(x):
    i = lax.bitcast_convert_type(x, jnp.int32)
    i = jnp.int32(0x5F3759DF) - (i >> 1)
    y = lax.bitcast_convert_type(i, jnp.float32)
    for _ in range(3):
        y = y * (jnp.float32(1.5) - jnp.float32(0.5) * x * y * y)
    return jnp.minimum(y, jnp.float32(1e12))


def _body(uidx_hbm, iidx_hbm, utab_hbm, itab_hbm, out_hbm,
          idx_u, idx_v, sidx_u, sidx_v, cols_u, cols_v, out_v, sem):
    wid = lax.axis_index("s") * NC + lax.axis_index("c")

    pltpu.sync_copy(uidx_hbm.at[pl.ds(wid * BPW, BPW)], idx_u)
    pltpu.sync_copy(iidx_hbm.at[pl.ds(wid * BPW, BPW)], idx_v)

    for idx, sidx in ((idx_u, sidx_u), (idx_v, sidx_v)):
        for c in range(NCHUNK):
            for k in range(CHUNK // L):
                v4 = idx[pl.ds(c * CHUNK + k * L, L)] << 2
                for j in range(D):
                    sidx[c * D + j, pl.ds(k * L, L)] = v4 + j

    copies = []
    for r in range(NROW):
        copies.append(pltpu.async_copy(
            utab_hbm.at[sidx_u.at[r]], cols_u.at[r], sem))
        copies.append(pltpu.async_copy(
            itab_hbm.at[sidx_v.at[r]], cols_v.at[r], sem))
    for c in copies:
        c.wait()

    for i in range(BPW // L):
        c = i // (CHUNK // L)
        o = (i % (CHUNK // L)) * L
        u = [cols_u[c * D + j, pl.ds(o, L)] for j in range(D)]
        v = [cols_v[c * D + j, pl.ds(o, L)] for j in range(D)]
        dot = u[0] * v[0] + u[1] * v[1] + u[2] * v[2] + u[3] * v[3]
        nu = u[0] * u[0] + u[1] * u[1] + u[2] * u[2] + u[3] * u[3]
        nv = v[0] * v[0] + v[1] * v[1] + v[2] * v[2] + v[3] * v[3]
        out_v[pl.ds(i * L, L)] = dot * _rsqrt_clamped(nu) * _rsqrt_clamped(nv)

    pltpu.sync_copy(out_v, out_hbm.at[pl.ds(wid * BPW, BPW)])


@jax.jit
def _run(user_input, item_input, user_table, item_table):
    utab = user_table.reshape(-1)
    itab = item_table.reshape(-1)
    mesh = plsc.VectorSubcoreMesh(core_axis_name="c", subcore_axis_name="s")
    f = pl.kernel(
        _body,
        out_type=jax.ShapeDtypeStruct((BATCH,), jnp.float32),
        mesh=mesh,
        scratch_types=[
            pltpu.VMEM((BPW,), jnp.int32),
            pltpu.VMEM((BPW,), jnp.int32),
            pltpu.VMEM((NROW, CHUNK), jnp.int32),
            pltpu.VMEM((NROW, CHUNK), jnp.int32),
            pltpu.VMEM((NROW, CHUNK), jnp.float32),
            pltpu.VMEM((NROW, CHUNK), jnp.float32),
            pltpu.VMEM((BPW,), jnp.float32),
            pltpu.SemaphoreType.DMA,
        ],
    )
    return f(user_input, item_input, utab, itab)


def kernel(user_input, item_input, user_table, item_table):
    return _run(user_input, item_input, user_table, item_table)
